# baseline (speedup 1.0000x reference)
import functools

import jax
import jax.numpy as jnp
from jax import lax
from jax.experimental import pallas as pl
from jax.experimental.pallas import tpu as pltpu
from jax.experimental.pallas import tpu_sc as plsc

N = 10000
E = 320000
D = 128
NUM_NODE_TYPES = 8
BN_EPS = 1e-5

NC = 2
NS = 16
K = 80
EPW = E // (NC * NS)
NCH = EPW // K
ZT = 10
ZR = N // ZT
NBUF = 5

_sc_mesh = plsc.VectorSubcoreMesh(
    core_axis_name="c", subcore_axis_name="s", num_cores=NC, num_subcores=NS)
_sc_params = pltpu.CompilerParams(needs_layout_passes=False)
_sc_params_nt = pltpu.CompilerParams(needs_layout_passes=False,
                                     use_tc_tiling_on_sc=False)


KL = 80
NCHL = EPW // KL
EPD = E // NS
NCHD = EPD // KL
NT = NUM_NODE_TYPES


def _newton_rsqrt(v):
    yi = jnp.int32(0x5F3759DF) - lax.shift_right_logical(
        plsc.bitcast(v, jnp.int32), 1)
    y = plsc.bitcast(yi, jnp.float32)
    for _ in range(3):
        y = y * (1.5 - 0.5 * v * y * y)
    return y


@functools.partial(
    pl.kernel,
    out_type=[
        jax.ShapeDtypeStruct((E,), jnp.float32),
        jax.ShapeDtypeStruct((NC * N * NT,), jnp.float32),
        jax.ShapeDtypeStruct((N,), jnp.float32),
    ],
    mesh=_sc_mesh,
    compiler_params=_sc_params,
    scratch_types=[
        pltpu.VMEM((EPD,), jnp.int32),
        pltpu.VMEM((EPD,), jnp.float32),
        pltpu.VMEM((N,), jnp.float32),
        pltpu.VMEM((N,), jnp.int32),
        pltpu.VMEM((EPW,), jnp.int32),
        pltpu.VMEM((EPW,), jnp.int32),
        pltpu.VMEM((EPW,), jnp.float32),
        pltpu.VMEM((EPW,), jnp.float32),
        [pltpu.VMEM((KL,), jnp.int32) for _ in range(NBUF)],
        [pltpu.VMEM((KL,), jnp.int32) for _ in range(NBUF)],
        pltpu.VMEM((N * NT // NS,), jnp.float32),
        pltpu.VMEM_SHARED((N,), jnp.float32),
        pltpu.VMEM_SHARED((N * NT,), jnp.float32),
        [pltpu.SemaphoreType.DMA for _ in range(NBUF)],
        [pltpu.SemaphoreType.DMA for _ in range(NBUF)],
    ],
)
def _l0_kernel(src_hbm, dst_hbm, ew_hbm, x_hbm, zeros8_hbm,
               c_hbm, s_hbm, dinv_hbm,
               ddst_v, dew_v, dinv_v, x_v, src_v, dst_v, ew_v, cout_v,
               didx_b, fidx_b, zb_v, dacc, acc, dsem, ssem):
    ci = lax.axis_index("c")
    si = lax.axis_index("s")
    base = ci * (NS * EPW) + si * EPW
    based = si * EPD
    pltpu.sync_copy(dst_hbm.at[pl.ds(based, EPD)], ddst_v)
    pltpu.sync_copy(ew_hbm.at[pl.ds(based, EPD)], dew_v)
    pltpu.sync_copy(x_hbm, x_v)
    pltpu.sync_copy(src_hbm.at[pl.ds(base, EPW)], src_v)
    pltpu.sync_copy(dst_hbm.at[pl.ds(base, EPW)], dst_v)
    pltpu.sync_copy(ew_hbm.at[pl.ds(base, EPW)], ew_v)

    sw = N * NT // NS
    pltpu.sync_copy(zeros8_hbm.at[pl.ds(si * sw, sw)], zb_v)
    pltpu.sync_copy(zb_v, acc.at[pl.ds(si * sw, sw)])

    @pl.when(si < ZT)
    def _():
        pltpu.sync_copy(zeros8_hbm.at[pl.ds(si * ZR, ZR)], zb_v.at[pl.ds(0, ZR)])
        pltpu.sync_copy(zb_v.at[pl.ds(0, ZR)], dacc.at[pl.ds(si * ZR, ZR)])

    plsc.subcore_barrier()

    def deg_outer(t, carry):
        for b in range(NBUF):
            jj = t * NBUF + b

            @pl.when(jj >= NBUF)
            def _():
                pltpu.make_async_copy(ew_hbm.at[pl.ds(0, KL)], didx_b[b],
                                      dsem[b]).wait()

            def dgrp(g, c2):
                didx_b[b][pl.ds(g * 16, 16)] = (
                    ddst_v[pl.ds(jj * KL + g * 16, 16)])
                return c2

            lax.fori_loop(0, KL // 16, dgrp, 0)
            pltpu.async_copy(dew_v.at[pl.ds(jj * KL, KL)],
                             dacc.at[didx_b[b]], dsem[b], add=True)
        return carry

    lax.fori_loop(0, NCHD // NBUF, deg_outer, 0)
    for b in range(NBUF):
        pltpu.make_async_copy(ew_hbm.at[pl.ds(0, KL)], didx_b[b],
                              dsem[b]).wait()
    plsc.subcore_barrier()

    pltpu.sync_copy(dacc, dinv_v)

    def rs(i, carry):
        dinv_v[pl.ds(i * 16, 16)] = _newton_rsqrt(
            dinv_v[pl.ds(i * 16, 16)] + 1.0)
        return carry

    lax.fori_loop(0, N // 16, rs, 0)

    def outer(t, carry):
        for b in range(NBUF):
            jj = t * NBUF + b

            @pl.when(jj >= NBUF)
            def _():
                pltpu.make_async_copy(ew_hbm.at[pl.ds(0, KL)], fidx_b[b],
                                      ssem[b]).wait()

            def grp(g, c2):
                off = jj * KL + g * 16
                sv = src_v[pl.ds(off, 16)]
                dv = dst_v[pl.ds(off, 16)]
                wv = ew_v[pl.ds(off, 16)]
                a = plsc.load_gather(dinv_v, [sv])
                bb = plsc.load_gather(dinv_v, [dv])
                cout_v[pl.ds(off, 16)] = a * bb * wv
                xs = plsc.load_gather(x_v, [sv])
                fidx_b[b][pl.ds(g * 16, 16)] = dv * NT + xs
                return c2

            lax.fori_loop(0, KL // 16, grp, 0)
            pltpu.async_copy(cout_v.at[pl.ds(jj * KL, KL)],
                             acc.at[fidx_b[b]], ssem[b], add=True)
        return carry

    lax.fori_loop(0, NCHL // NBUF, outer, 0)
    for b in range(NBUF):
        pltpu.make_async_copy(ew_hbm.at[pl.ds(0, KL)], fidx_b[b],
                              ssem[b]).wait()
    pltpu.sync_copy(cout_v, c_hbm.at[pl.ds(base, EPW)])

    @pl.when((si < ZT) & (ci == 0))
    def _():
        pltpu.sync_copy(dinv_v.at[pl.ds(si * ZR, ZR)],
                        dinv_hbm.at[pl.ds(si * ZR, ZR)])

    plsc.subcore_barrier()
    pltpu.sync_copy(acc.at[pl.ds(si * sw, sw)], zb_v)
    pltpu.sync_copy(zb_v, s_hbm.at[pl.ds(ci * (N * NT) + si * sw, sw)])


KA = 40
NCHA = EPW // KA
NSTG = 2
_BCAST_DN = lax.GatherDimensionNumbers(
    offset_dims=(), collapsed_slice_dims=(0,), start_index_map=(0,))


def _lane_bcast(vec, e16):
    idx = jnp.full((16, 1), e16, jnp.int32)
    return lax.gather(vec, idx, _BCAST_DN, slice_sizes=(1,),
                      mode=lax.GatherScatterMode.PROMISE_IN_BOUNDS)


@functools.partial(
    pl.kernel,
    out_type=jax.ShapeDtypeStruct((NC, N, D), jnp.float32),
    mesh=_sc_mesh,
    compiler_params=_sc_params_nt,
    scratch_types=[
        pltpu.VMEM((EPW,), jnp.int32),
        [pltpu.VMEM((KA, D // 2), jnp.int32) for _ in range(NBUF)],
        [pltpu.VMEM((48,), jnp.int32) for _ in range(NBUF)],
        [pltpu.VMEM((48,), jnp.float32) for _ in range(NBUF)],
        [pltpu.VMEM((KA, D), jnp.float32) for _ in range(NSTG)],
        [pltpu.VMEM((KA,), jnp.int32) for _ in range(NSTG)],
        pltpu.VMEM_SHARED((N, D), jnp.float32),
        [pltpu.SemaphoreType.DMA for _ in range(NBUF)],
        [pltpu.SemaphoreType.DMA for _ in range(NBUF)],
        [pltpu.SemaphoreType.DMA for _ in range(NSTG)],
    ],
)
def _agg_kernel(g_hbm, src_hbm, dst_hbm, c_hbm, zeros_hbm, out_hbm,
                src_v, rows_b, didx_b, cbuf_b, stg_b, sidx_b, acc,
                gsem, dsem, ssem):
    ci = lax.axis_index("c")
    si = lax.axis_index("s")
    base = ci * (NS * EPW) + si * EPW
    pltpu.sync_copy(src_hbm.at[pl.ds(base, EPW)], src_v)

    @pl.when(si < ZT)
    def _():
        pltpu.sync_copy(zeros_hbm.at[pl.ds(si * ZR, ZR), :],
                        acc.at[pl.ds(si * ZR, ZR), :])

    plsc.subcore_barrier()

    def issue_pair(b, j):
        off = pl.multiple_of(base + j * KA, 8)
        pltpu.async_copy(dst_hbm.at[pl.ds(off, KA)],
                         didx_b[b].at[pl.ds(0, KA)], dsem[b])
        pltpu.async_copy(c_hbm.at[pl.ds(off, KA)],
                         cbuf_b[b].at[pl.ds(0, KA)], dsem[b])

    def issue_gather(b, j):
        pltpu.async_copy(g_hbm.at[src_v.at[pl.ds(j * KA, KA)]], rows_b[b],
                         gsem[b])

    def wait_pair(b):
        pltpu.make_async_copy(dst_hbm.at[pl.ds(0, KA)],
                              didx_b[b].at[pl.ds(0, KA)], dsem[b]).wait()
        pltpu.make_async_copy(c_hbm.at[pl.ds(0, KA)],
                              cbuf_b[b].at[pl.ds(0, KA)], dsem[b]).wait()

    def wait_gather(b):
        pltpu.make_async_copy(g_hbm.at[pl.ds(0, KA), :], rows_b[b],
                              gsem[b]).wait()

    def wait_scatter(r):
        pltpu.make_async_copy(zeros_hbm.at[pl.ds(0, KA), :], stg_b[r],
                              ssem[r]).wait()

    for b in range(NBUF):
        issue_pair(b, b)
        issue_gather(b, b)

    hi_mask = jnp.full((16,), -65536, jnp.int32)

    def outer(t, carry):
        for b in range(NBUF):
            jj = t * NBUF + b
            r = b % NSTG if NBUF % NSTG == 0 else None
        return carry

    def outer10(t, carry):
        for u in range(2 * NBUF):
            b = u % NBUF
            r = u % NSTG
            jj = t * (2 * NBUF) + u
            wait_pair(b)
            wait_gather(b)

            @pl.when(jj >= NSTG)
            def _():
                wait_scatter(r)

            for o in (0, 16, 24):
                sidx_b[r][pl.ds(o, 16)] = didx_b[b][pl.ds(o, 16)]

            def grp(g, c2):
                cg = cbuf_b[b][pl.ds(g * 8, 16)]
                for e16 in range(8):
                    cvec = _lane_bcast(cg, e16)
                    e = g * 8 + e16
                    for kk in range(D // 32):
                        wi = rows_b[b][e, pl.ds(kk * 16, 16)]
                        ev = plsc.bitcast(
                            lax.shift_left(wi, 16), jnp.float32) * cvec
                        od = plsc.bitcast(wi & hi_mask, jnp.float32) * cvec
                        stg_b[r][e, pl.ds(kk * 16, 16)] = ev
                        stg_b[r][e, pl.ds(D // 2 + kk * 16, 16)] = od
                return c2

            lax.fori_loop(0, KA // 8, grp, 0)
            pltpu.async_copy(stg_b[r], acc.at[sidx_b[r]], ssem[r], add=True)
            j5 = jj + NBUF

            @pl.when(j5 <= NCHA - 1)
            def _():
                issue_pair(b, j5)
                issue_gather(b, j5)
        return carry

    lax.fori_loop(0, NCHA // (2 * NBUF), outer10, 0)
    wait_scatter(0)
    wait_scatter(1)
    plsc.subcore_barrier()

    @pl.when(si < ZT)
    def _():
        pltpu.sync_copy(acc.at[pl.ds(si * ZR, ZR), :],
                        out_hbm.at[ci, pl.ds(si * ZR, ZR), :])


def _mid_body(s_ref, x_ref, emb_ref, w0_ref, dinv_ref, b0_ref, g0_ref,
              be0_ref, w1_ref, hw1_ref):
    dinv = dinv_ref[...]
    t0 = jnp.dot(emb_ref[...], w0_ref[...], preferred_element_type=jnp.float32)
    onehot = (lax.broadcasted_iota(jnp.int32, (N, NUM_NODE_TYPES), 1)
              == x_ref[...][:, None]).astype(jnp.float32)
    s = s_ref[0] + s_ref[1] + (dinv * dinv)[:, None] * onehot
    h = b0_ref[...][None, :] + s[:, 0:1] * t0[0][None, :]
    for t in range(1, NUM_NODE_TYPES):
        h = h + s[:, t:t + 1] * t0[t][None, :]
    m = jnp.mean(h, axis=0)
    v = jnp.mean(h * h, axis=0) - m * m
    hbn = ((h - m[None, :]) * lax.rsqrt(v + BN_EPS) * g0_ref[...][None, :]
           + be0_ref[...][None, :])
    hr = jnp.maximum(hbn, 0.0)
    hw1_ref[...] = jnp.dot(hr, w1_ref[...], preferred_element_type=jnp.float32)


def _fin_body(agg_ref, hw1_ref, dinv_ref, b1_ref, g1_ref, be1_ref, out_ref):
    dinv = dinv_ref[...]
    h = (agg_ref[0] + agg_ref[1]
         + (dinv * dinv)[:, None] * hw1_ref[...] + b1_ref[...][None, :])
    m = jnp.mean(h, axis=0)
    v = jnp.mean(h * h, axis=0) - m * m
    out_ref[...] = ((h - m[None, :]) * lax.rsqrt(v + BN_EPS)
                    * g1_ref[...][None, :] + be1_ref[...][None, :])


def kernel(x, edge_index, edge_attr, batch, emb, W0, b0, g0, be0,
           W1, b1, g1, be1):
    del batch
    src = edge_index[0].astype(jnp.int32)
    dst = edge_index[1].astype(jnp.int32)
    x = x.astype(jnp.int32)
    ew = edge_attr.astype(jnp.float32)
    zeros2 = jnp.zeros((N, D), jnp.float32)
    zeros8 = jnp.zeros((N * NUM_NODE_TYPES,), jnp.float32)

    c, s_part, dinv = _l0_kernel(src, dst, ew, x, zeros8)
    s_part = s_part.reshape(NC, N, NUM_NODE_TYPES)

    hw1 = pl.pallas_call(
        _mid_body,
        out_shape=jax.ShapeDtypeStruct((N, D), jnp.float32),
    )(s_part, x, emb, W0, dinv, b0, g0, be0, W1)

    hswz = jnp.stack([hw1[:, :D // 2], hw1[:, D // 2:]],
                     axis=-1).astype(jnp.bfloat16)
    gi32 = lax.bitcast_convert_type(hswz, jnp.int32)
    agg1 = _agg_kernel(gi32, src, dst, c, zeros2)

    out = pl.pallas_call(
        _fin_body,
        out_shape=jax.ShapeDtypeStruct((N, D), jnp.float32),
    )(agg1, hw1, dinv, b1, g1, be1)
    return out


# speedup vs baseline: 1.5742x; 1.5742x over previous
import functools

import jax
import jax.numpy as jnp
from jax import lax
from jax.experimental import pallas as pl
from jax.experimental.pallas import tpu as pltpu
from jax.experimental.pallas import tpu_sc as plsc

N = 10000
E = 320000
D = 128
NUM_NODE_TYPES = 8
BN_EPS = 1e-5

NC = 2
NS = 16
K = 80
EPW = E // (NC * NS)
NCH = EPW // K
ZT = 10
ZR = N // ZT
NBUF = 5

_sc_mesh = plsc.VectorSubcoreMesh(
    core_axis_name="c", subcore_axis_name="s", num_cores=NC, num_subcores=NS)
_sc_params = pltpu.CompilerParams(needs_layout_passes=False)
_sc_params_nt = pltpu.CompilerParams(needs_layout_passes=False,
                                     use_tc_tiling_on_sc=False)


KL = 80
NCHL = EPW // KL
EPD = E // NS
NCHD = EPD // KL
NT = NUM_NODE_TYPES


def _newton_rsqrt(v):
    yi = jnp.int32(0x5F3759DF) - lax.shift_right_logical(
        plsc.bitcast(v, jnp.int32), 1)
    y = plsc.bitcast(yi, jnp.float32)
    for _ in range(3):
        y = y * (1.5 - 0.5 * v * y * y)
    return y


@functools.partial(
    pl.kernel,
    out_type=[
        jax.ShapeDtypeStruct((E,), jnp.float32),
        jax.ShapeDtypeStruct((NC * N * NT,), jnp.float32),
        jax.ShapeDtypeStruct((N,), jnp.float32),
    ],
    mesh=_sc_mesh,
    compiler_params=_sc_params,
    scratch_types=[
        pltpu.VMEM((EPD,), jnp.int32),
        pltpu.VMEM((EPD,), jnp.float32),
        pltpu.VMEM((N,), jnp.float32),
        pltpu.VMEM((N,), jnp.int32),
        pltpu.VMEM((EPW,), jnp.int32),
        pltpu.VMEM((EPW,), jnp.int32),
        pltpu.VMEM((EPW,), jnp.float32),
        pltpu.VMEM((EPW,), jnp.float32),
        [pltpu.VMEM((KL,), jnp.int32) for _ in range(NBUF)],
        [pltpu.VMEM((KL,), jnp.int32) for _ in range(NBUF)],
        pltpu.VMEM((N * NT // NS,), jnp.float32),
        pltpu.VMEM_SHARED((N,), jnp.float32),
        pltpu.VMEM_SHARED((N * NT,), jnp.float32),
        [pltpu.SemaphoreType.DMA for _ in range(NBUF)],
        [pltpu.SemaphoreType.DMA for _ in range(NBUF)],
    ],
)
def _l0_kernel(src_hbm, dst_hbm, ew_hbm, x_hbm, zeros8_hbm,
               c_hbm, s_hbm, dinv_hbm,
               ddst_v, dew_v, dinv_v, x_v, src_v, dst_v, ew_v, cout_v,
               didx_b, fidx_b, zb_v, dacc, acc, dsem, ssem):
    ci = lax.axis_index("c")
    si = lax.axis_index("s")
    base = ci * (NS * EPW) + si * EPW
    based = si * EPD
    pltpu.sync_copy(dst_hbm.at[pl.ds(based, EPD)], ddst_v)
    pltpu.sync_copy(ew_hbm.at[pl.ds(based, EPD)], dew_v)
    pltpu.sync_copy(x_hbm, x_v)
    pltpu.sync_copy(src_hbm.at[pl.ds(base, EPW)], src_v)
    pltpu.sync_copy(dst_hbm.at[pl.ds(base, EPW)], dst_v)
    pltpu.sync_copy(ew_hbm.at[pl.ds(base, EPW)], ew_v)

    sw = N * NT // NS
    pltpu.sync_copy(zeros8_hbm.at[pl.ds(si * sw, sw)], zb_v)
    pltpu.sync_copy(zb_v, acc.at[pl.ds(si * sw, sw)])

    @pl.when(si < ZT)
    def _():
        pltpu.sync_copy(zeros8_hbm.at[pl.ds(si * ZR, ZR)], zb_v.at[pl.ds(0, ZR)])
        pltpu.sync_copy(zb_v.at[pl.ds(0, ZR)], dacc.at[pl.ds(si * ZR, ZR)])

    plsc.subcore_barrier()

    def deg_outer(t, carry):
        for b in range(NBUF):
            jj = t * NBUF + b

            @pl.when(jj >= NBUF)
            def _():
                pltpu.make_async_copy(ew_hbm.at[pl.ds(0, KL)], didx_b[b],
                                      dsem[b]).wait()

            def dgrp(g, c2):
                didx_b[b][pl.ds(g * 16, 16)] = (
                    ddst_v[pl.ds(jj * KL + g * 16, 16)])
                return c2

            lax.fori_loop(0, KL // 16, dgrp, 0)
            pltpu.async_copy(dew_v.at[pl.ds(jj * KL, KL)],
                             dacc.at[didx_b[b]], dsem[b], add=True)
        return carry

    lax.fori_loop(0, NCHD // NBUF, deg_outer, 0)
    for b in range(NBUF):
        pltpu.make_async_copy(ew_hbm.at[pl.ds(0, KL)], didx_b[b],
                              dsem[b]).wait()
    plsc.subcore_barrier()

    pltpu.sync_copy(dacc, dinv_v)

    def rs(i, carry):
        dinv_v[pl.ds(i * 16, 16)] = _newton_rsqrt(
            dinv_v[pl.ds(i * 16, 16)] + 1.0)
        return carry

    lax.fori_loop(0, N // 16, rs, 0)

    def outer(t, carry):
        for b in range(NBUF):
            jj = t * NBUF + b

            @pl.when(jj >= NBUF)
            def _():
                pltpu.make_async_copy(ew_hbm.at[pl.ds(0, KL)], fidx_b[b],
                                      ssem[b]).wait()

            def grp(g, c2):
                off = jj * KL + g * 16
                sv = src_v[pl.ds(off, 16)]
                dv = dst_v[pl.ds(off, 16)]
                wv = ew_v[pl.ds(off, 16)]
                a = plsc.load_gather(dinv_v, [sv])
                bb = plsc.load_gather(dinv_v, [dv])
                cout_v[pl.ds(off, 16)] = a * bb * wv
                xs = plsc.load_gather(x_v, [sv])
                fidx_b[b][pl.ds(g * 16, 16)] = dv * NT + xs
                return c2

            lax.fori_loop(0, KL // 16, grp, 0)
            pltpu.async_copy(cout_v.at[pl.ds(jj * KL, KL)],
                             acc.at[fidx_b[b]], ssem[b], add=True)
        return carry

    lax.fori_loop(0, NCHL // NBUF, outer, 0)
    for b in range(NBUF):
        pltpu.make_async_copy(ew_hbm.at[pl.ds(0, KL)], fidx_b[b],
                              ssem[b]).wait()
    pltpu.sync_copy(cout_v, c_hbm.at[pl.ds(base, EPW)])

    @pl.when((si < ZT) & (ci == 0))
    def _():
        pltpu.sync_copy(dinv_v.at[pl.ds(si * ZR, ZR)],
                        dinv_hbm.at[pl.ds(si * ZR, ZR)])

    plsc.subcore_barrier()
    pltpu.sync_copy(acc.at[pl.ds(si * sw, sw)], zb_v)
    pltpu.sync_copy(zb_v, s_hbm.at[pl.ds(ci * (N * NT) + si * sw, sw)])


KA = 40
NCHA = EPW // KA
_BCAST_DN = lax.GatherDimensionNumbers(
    offset_dims=(), collapsed_slice_dims=(0,), start_index_map=(0,))


def _lane_bcast(vec, e16):
    idx = jnp.full((16, 1), e16, jnp.int32)
    return lax.gather(vec, idx, _BCAST_DN, slice_sizes=(1,),
                      mode=lax.GatherScatterMode.PROMISE_IN_BOUNDS)


@functools.partial(
    pl.kernel,
    out_type=jax.ShapeDtypeStruct((NC, N, D), jnp.float32),
    mesh=_sc_mesh,
    compiler_params=_sc_params,
    scratch_types=[
        pltpu.VMEM((EPW,), jnp.int32),
        [pltpu.VMEM((KA, D), jnp.float32) for _ in range(NBUF)],
        [pltpu.VMEM((KA,), jnp.int32) for _ in range(NBUF)],
        [pltpu.VMEM((48,), jnp.float32) for _ in range(NBUF)],
        pltpu.VMEM_SHARED((N, D), jnp.float32),
        [pltpu.SemaphoreType.DMA for _ in range(NBUF)],
        [pltpu.SemaphoreType.DMA for _ in range(NBUF)],
        [pltpu.SemaphoreType.DMA for _ in range(NBUF)],
    ],
)
def _agg_kernel(g_hbm, src_hbm, dst_hbm, c_hbm, zeros_hbm, out_hbm,
                src_v, rows_b, didx_b, cbuf_b, acc, gsem, dsem, ssem):
    ci = lax.axis_index("c")
    si = lax.axis_index("s")
    base = ci * (NS * EPW) + si * EPW
    pltpu.sync_copy(src_hbm.at[pl.ds(base, EPW)], src_v)

    @pl.when(si < ZT)
    def _():
        pltpu.sync_copy(zeros_hbm.at[pl.ds(si * ZR, ZR), :],
                        acc.at[pl.ds(si * ZR, ZR), :])

    plsc.subcore_barrier()

    def issue_pair(b, j):
        off = pl.multiple_of(base + j * KA, 8)
        pltpu.async_copy(dst_hbm.at[pl.ds(off, KA)], didx_b[b], dsem[b])
        pltpu.async_copy(c_hbm.at[pl.ds(off, KA)],
                         cbuf_b[b].at[pl.ds(0, KA)], dsem[b])

    def issue_gather(b, j):
        pltpu.async_copy(g_hbm.at[src_v.at[pl.ds(j * KA, KA)]], rows_b[b],
                         gsem[b])

    def wait_pair(b):
        pltpu.make_async_copy(dst_hbm.at[pl.ds(0, KA)], didx_b[b],
                              dsem[b]).wait()
        pltpu.make_async_copy(c_hbm.at[pl.ds(0, KA)],
                              cbuf_b[b].at[pl.ds(0, KA)], dsem[b]).wait()

    def wait_rows(b, sem):
        pltpu.make_async_copy(zeros_hbm.at[pl.ds(0, KA), :], rows_b[b],
                              sem[b]).wait()

    for b in range(3):
        issue_pair(b, b)
        issue_gather(b, b)

    def outer(t, carry):
        for b in range(NBUF):
            jj = t * NBUF + b
            wait_pair(b)
            wait_rows(b, gsem)

            def grp(g, c2):
                cg = cbuf_b[b][pl.ds(g * 8, 16)]
                for e16 in range(8):
                    cvec = _lane_bcast(cg, e16)
                    e = g * 8 + e16
                    for k in range(D // 16):
                        rows_b[b][e, pl.ds(k * 16, 16)] = (
                            rows_b[b][e, pl.ds(k * 16, 16)] * cvec)
                return c2

            lax.fori_loop(0, KA // 8, grp, 0)
            pltpu.async_copy(rows_b[b], acc.at[didx_b[b]], ssem[b], add=True)
            b2 = (b - 2) % NBUF
            j3 = jj + 3

            @pl.when(jj >= 2)
            def _():
                wait_rows(b2, ssem)

            @pl.when(j3 <= NCHA - 1)
            def _():
                issue_pair(b2, j3)
                issue_gather(b2, j3)
        return carry

    lax.fori_loop(0, NCHA // NBUF, outer, 0)
    wait_rows((NCHA - 2) % NBUF, ssem)
    wait_rows((NCHA - 1) % NBUF, ssem)
    plsc.subcore_barrier()

    @pl.when(si < ZT)
    def _():
        pltpu.sync_copy(acc.at[pl.ds(si * ZR, ZR), :],
                        out_hbm.at[ci, pl.ds(si * ZR, ZR), :])


def _mid_body(s_ref, x_ref, emb_ref, w0_ref, dinv_ref, b0_ref, g0_ref,
              be0_ref, w1_ref, hw1_ref):
    dinv = dinv_ref[...]
    t0 = jnp.dot(emb_ref[...], w0_ref[...], preferred_element_type=jnp.float32)
    onehot = (lax.broadcasted_iota(jnp.int32, (N, NUM_NODE_TYPES), 1)
              == x_ref[...][:, None]).astype(jnp.float32)
    s = s_ref[0] + s_ref[1] + (dinv * dinv)[:, None] * onehot
    h = b0_ref[...][None, :] + s[:, 0:1] * t0[0][None, :]
    for t in range(1, NUM_NODE_TYPES):
        h = h + s[:, t:t + 1] * t0[t][None, :]
    m = jnp.mean(h, axis=0)
    v = jnp.mean(h * h, axis=0) - m * m
    hbn = ((h - m[None, :]) * lax.rsqrt(v + BN_EPS) * g0_ref[...][None, :]
           + be0_ref[...][None, :])
    hr = jnp.maximum(hbn, 0.0)
    hw1_ref[...] = jnp.dot(hr, w1_ref[...], preferred_element_type=jnp.float32)


def _fin_body(agg_ref, hw1_ref, dinv_ref, b1_ref, g1_ref, be1_ref, out_ref):
    dinv = dinv_ref[...]
    h = (agg_ref[0] + agg_ref[1]
         + (dinv * dinv)[:, None] * hw1_ref[...] + b1_ref[...][None, :])
    m = jnp.mean(h, axis=0)
    v = jnp.mean(h * h, axis=0) - m * m
    out_ref[...] = ((h - m[None, :]) * lax.rsqrt(v + BN_EPS)
                    * g1_ref[...][None, :] + be1_ref[...][None, :])


def kernel(x, edge_index, edge_attr, batch, emb, W0, b0, g0, be0,
           W1, b1, g1, be1):
    del batch
    src = edge_index[0].astype(jnp.int32)
    dst = edge_index[1].astype(jnp.int32)
    x = x.astype(jnp.int32)
    ew = edge_attr.astype(jnp.float32)
    zeros2 = jnp.zeros((N, D), jnp.float32)
    zeros8 = jnp.zeros((N * NUM_NODE_TYPES,), jnp.float32)

    c, s_part, dinv = _l0_kernel(src, dst, ew, x, zeros8)
    s_part = s_part.reshape(NC, N, NUM_NODE_TYPES)

    hw1 = pl.pallas_call(
        _mid_body,
        out_shape=jax.ShapeDtypeStruct((N, D), jnp.float32),
    )(s_part, x, emb, W0, dinv, b0, g0, be0, W1)

    agg1 = _agg_kernel(hw1, src, dst, c, zeros2)

    out = pl.pallas_call(
        _fin_body,
        out_shape=jax.ShapeDtypeStruct((N, D), jnp.float32),
    )(agg1, hw1, dinv, b1, g1, be1)
    return out
